# baseline (speedup 1.0000x reference)
import jax
import jax.numpy as jnp
import numpy as np
from jax import lax
from jax.experimental import pallas as pl
from jax.experimental.pallas import tpu as pltpu
from jax.experimental.pallas import tpu_sc as plsc

B, F, D = 16384, 100, 1000000
NC, NS = 2, 16
NW = NC * NS
S = B // NW
K = F * S
LANES = 16
NCHK = 8
CS = S // NCHK
CE = CS * F
SB = CS // LANES
CE_H = 2176
CE_S = CE - CE_H

_Q = np.arange(CE)
_PTAB = (((_Q // F) << 7) | (_Q % F)).astype(np.int32)


def _wide_body(idx_hbm, w_hbm, b_hbm, ptab_hbm, out_hbm,
               idx2d_v, idx_v, vals_v, out_v, b_v, ptab_v,
               w_sh, isem0, isem1, hsem0, hsem1, ssem0, ssem1, wsem, tsem):
    c = lax.axis_index("c")
    s = lax.axis_index("s")
    wid = s * NC + c
    row0 = wid * S

    wcopy = pltpu.make_async_copy(w_hbm, w_sh, wsem)

    @pl.when(s == 0)
    def _():
        wcopy.start()

    pltpu.async_copy(ptab_hbm, ptab_v, tsem).wait()

    isems = (isem0, isem1)
    hsems = (hsem0, hsem1)
    ssems = (ssem0, ssem1)

    def fire_stage(chunk):
        p = chunk % 2
        return pltpu.async_copy(
            idx_hbm.at[pl.ds(row0 + chunk * CS, CS), :],
            idx2d_v.at[pl.ds(p * CS, CS), :], isems[p])

    def flatten(chunk):
        p = chunk % 2

        def body(j, _):
            for u in range(2):
                o = (2 * j + u) * LANES
                pt = ptab_v[pl.ds(o, LANES)]
                rows = lax.shift_right_logical(pt, 7) + p * CS
                cols = lax.bitwise_and(pt, 127)
                v = plsc.load_gather(idx2d_v, [rows, cols])
                idx_v[pl.ds(p * CE + o, LANES)] = v
            return 0

        lax.fori_loop(0, CE // LANES // 2, body, 0)

    def fire_h(chunk):
        p = chunk % 2
        return pltpu.async_copy(
            w_hbm.at[idx_v.at[pl.ds(p * CE, CE_H)]],
            vals_v.at[pl.ds(p * CE, CE_H)], hsems[p])

    def fire_s(chunk):
        p = chunk % 2
        return pltpu.async_copy(
            w_sh.at[idx_v.at[pl.ds(p * CE + CE_H, CE_S)]],
            vals_v.at[pl.ds(p * CE + CE_H, CE_S)], ssems[p])

    lane_f = lax.iota(jnp.int32, LANES) * F

    def reduce_chunk(chunk):
        p = chunk % 2

        def body(f, carry):
            accs, idxvs = carry
            accs = tuple(accs[i] + plsc.load_gather(vals_v, [idxvs[i]])
                         for i in range(SB))
            idxvs = tuple(iv + 1 for iv in idxvs)
            return (accs, idxvs)

        init = (tuple(jnp.zeros((LANES,), jnp.float32) for _ in range(SB)),
                tuple(lane_f + (p * CS + i * LANES) * F for i in range(SB)))
        accs, _ = lax.fori_loop(0, F, body, init)
        bvec = b_v[...]
        for b_i in range(SB):
            z = accs[b_i] + bvec
            z = jnp.clip(z, -35.0, 35.0)
            y = 1.0 / (1.0 + jnp.exp(-z))
            out_v[pl.ds(chunk * CS + b_i * LANES, LANES)] = y

    pltpu.sync_copy(b_hbm, b_v)
    stage_pend = [fire_stage(0), fire_stage(1)]
    stage_pend[0].wait()
    flatten(0)
    stage_pend[0] = fire_stage(2)
    gh0 = fire_h(0)
    stage_pend[1].wait()
    flatten(1)
    stage_pend[1] = fire_stage(3)
    gh1 = fire_h(1)

    @pl.when(s == 0)
    def _():
        wcopy.wait()
    plsc.subcore_barrier()

    g_pend = [(gh0, fire_s(0)), (gh1, fire_s(1))]
    for chunk in range(NCHK):
        p = chunk % 2
        for d in g_pend[p]:
            d.wait()
        if chunk + 2 < NCHK:
            stage_pend[p].wait()
            flatten(chunk + 2)
        reduce_chunk(chunk)
        if chunk + 2 < NCHK:
            if chunk + 4 < NCHK:
                stage_pend[p] = fire_stage(chunk + 4)
            g_pend[p] = (fire_h(chunk + 2), fire_s(chunk + 2))

    pltpu.sync_copy(out_v, out_hbm.at[pl.ds(wid * S, S)])


@jax.jit
def _wide_forward(idx, w, b_arr):
    mesh = plsc.VectorSubcoreMesh(core_axis_name="c", subcore_axis_name="s")
    return pl.kernel(
        _wide_body,
        out_type=jax.ShapeDtypeStruct((B,), jnp.float32),
        mesh=mesh,
        scratch_types=[
            pltpu.VMEM((2 * CS, F), jnp.int32),
            pltpu.VMEM((2 * CE,), jnp.int32),
            pltpu.VMEM((2 * CE,), jnp.float32),
            pltpu.VMEM((S,), jnp.float32),
            pltpu.VMEM((LANES,), jnp.float32),
            pltpu.VMEM((CE,), jnp.int32),
            pltpu.VMEM_SHARED((D,), jnp.float32),
            pltpu.SemaphoreType.DMA,
            pltpu.SemaphoreType.DMA,
            pltpu.SemaphoreType.DMA,
            pltpu.SemaphoreType.DMA,
            pltpu.SemaphoreType.DMA,
            pltpu.SemaphoreType.DMA,
            pltpu.SemaphoreType.DMA,
            pltpu.SemaphoreType.DMA,
        ],
        compiler_params=pltpu.CompilerParams(
            needs_layout_passes=False, use_tc_tiling_on_sc=True),
    )(idx, w, b_arr, jnp.asarray(_PTAB))


def kernel(X_w_indices, X_d, y_pred, y, w, b):
    idx = X_w_indices.astype(jnp.int32)
    b_arr = jnp.broadcast_to(b.astype(jnp.float32), (LANES,))
    return _wide_forward(idx, w, b_arr)


# speedup vs baseline: 1.0194x; 1.0194x over previous
import jax
import jax.numpy as jnp
import numpy as np
from jax import lax
from jax.experimental import pallas as pl
from jax.experimental.pallas import tpu as pltpu
from jax.experimental.pallas import tpu_sc as plsc

B, F, D = 16384, 100, 1000000
NC, NS = 2, 16
NW = NC * NS
S = B // NW
K = F * S
LANES = 16
NCHK = 8
CS = S // NCHK
CE = CS * F
SB = CS // LANES
CE_H = 1792
CE_S = CE - CE_H

_Q = np.arange(CE)
_PTAB = (((_Q // F) << 7) | (_Q % F)).astype(np.int32)


def _wide_body(idx_hbm, w_hbm, b_hbm, ptab_hbm, out_hbm,
               idx2d_v, idx_v, vals_v, out_v, b_v, ptab_v,
               w_sh, isem0, isem1, hsem0, hsem1, ssem0, ssem1, wsem, tsem):
    c = lax.axis_index("c")
    s = lax.axis_index("s")
    wid = s * NC + c
    row0 = wid * S

    wcopy = pltpu.make_async_copy(w_hbm, w_sh, wsem)

    @pl.when(s == 0)
    def _():
        wcopy.start()

    pltpu.async_copy(ptab_hbm, ptab_v, tsem).wait()

    isems = (isem0, isem1)
    hsems = (hsem0, hsem1)
    ssems = (ssem0, ssem1)

    def fire_stage(chunk):
        p = chunk % 2
        return pltpu.async_copy(
            idx_hbm.at[pl.ds(row0 + chunk * CS, CS), :],
            idx2d_v.at[pl.ds(p * CS, CS), :], isems[p])

    def flatten(chunk):
        p = chunk % 2

        def body(j, _):
            for u in range(2):
                o = (2 * j + u) * LANES
                pt = ptab_v[pl.ds(o, LANES)]
                rows = lax.shift_right_logical(pt, 7) + p * CS
                cols = lax.bitwise_and(pt, 127)
                v = plsc.load_gather(idx2d_v, [rows, cols])
                idx_v[pl.ds(p * CE + o, LANES)] = v
            return 0

        lax.fori_loop(0, CE // LANES // 2, body, 0)

    def fire_h(chunk):
        p = chunk % 2
        return pltpu.async_copy(
            w_hbm.at[idx_v.at[pl.ds(p * CE, CE_H)]],
            vals_v.at[pl.ds(p * CE, CE_H)], hsems[p])

    def fire_s(chunk):
        p = chunk % 2
        return pltpu.async_copy(
            w_sh.at[idx_v.at[pl.ds(p * CE + CE_H, CE_S)]],
            vals_v.at[pl.ds(p * CE + CE_H, CE_S)], ssems[p])

    lane_f = lax.iota(jnp.int32, LANES) * F

    def reduce_chunk(chunk):
        p = chunk % 2

        def body(f, carry):
            accs, idxvs = carry
            accs = tuple(accs[i] + plsc.load_gather(vals_v, [idxvs[i]])
                         for i in range(SB))
            idxvs = tuple(iv + 1 for iv in idxvs)
            return (accs, idxvs)

        init = (tuple(jnp.zeros((LANES,), jnp.float32) for _ in range(SB)),
                tuple(lane_f + (p * CS + i * LANES) * F for i in range(SB)))
        accs, _ = lax.fori_loop(0, F, body, init)
        bvec = b_v[...]
        for b_i in range(SB):
            z = accs[b_i] + bvec
            z = jnp.clip(z, -35.0, 35.0)
            y = 1.0 / (1.0 + jnp.exp(-z))
            out_v[pl.ds(chunk * CS + b_i * LANES, LANES)] = y

    pltpu.sync_copy(b_hbm, b_v)
    stage_pend = [fire_stage(0), fire_stage(1)]
    stage_pend[0].wait()
    flatten(0)
    gh0 = fire_h(0)

    @pl.when(s == 0)
    def _():
        wcopy.wait()
    plsc.subcore_barrier()

    g_pend = [None, None]
    g_pend[0] = (gh0, fire_s(0))
    for chunk in range(NCHK):
        p = chunk % 2
        if chunk + 1 < NCHK:
            stage_pend[(chunk + 1) % 2].wait()
            flatten(chunk + 1)
            if chunk + 2 < NCHK:
                stage_pend[p] = fire_stage(chunk + 2)
            g_pend[(chunk + 1) % 2] = (fire_h(chunk + 1), fire_s(chunk + 1))
        for d in g_pend[p]:
            d.wait()
        reduce_chunk(chunk)

    pltpu.sync_copy(out_v, out_hbm.at[pl.ds(wid * S, S)])


@jax.jit
def _wide_forward(idx, w, b_arr):
    mesh = plsc.VectorSubcoreMesh(core_axis_name="c", subcore_axis_name="s")
    return pl.kernel(
        _wide_body,
        out_type=jax.ShapeDtypeStruct((B,), jnp.float32),
        mesh=mesh,
        scratch_types=[
            pltpu.VMEM((2 * CS, F), jnp.int32),
            pltpu.VMEM((2 * CE,), jnp.int32),
            pltpu.VMEM((2 * CE,), jnp.float32),
            pltpu.VMEM((S,), jnp.float32),
            pltpu.VMEM((LANES,), jnp.float32),
            pltpu.VMEM((CE,), jnp.int32),
            pltpu.VMEM_SHARED((D,), jnp.float32),
            pltpu.SemaphoreType.DMA,
            pltpu.SemaphoreType.DMA,
            pltpu.SemaphoreType.DMA,
            pltpu.SemaphoreType.DMA,
            pltpu.SemaphoreType.DMA,
            pltpu.SemaphoreType.DMA,
            pltpu.SemaphoreType.DMA,
            pltpu.SemaphoreType.DMA,
        ],
        compiler_params=pltpu.CompilerParams(
            needs_layout_passes=False, use_tc_tiling_on_sc=True),
    )(idx, w, b_arr, jnp.asarray(_PTAB))


def kernel(X_w_indices, X_d, y_pred, y, w, b):
    idx = X_w_indices.astype(jnp.int32)
    b_arr = jnp.broadcast_to(b.astype(jnp.float32), (LANES,))
    return _wide_forward(idx, w, b_arr)


# speedup vs baseline: 1.0243x; 1.0048x over previous
import jax
import jax.numpy as jnp
import numpy as np
from jax import lax
from jax.experimental import pallas as pl
from jax.experimental.pallas import tpu as pltpu
from jax.experimental.pallas import tpu_sc as plsc

B, F, D = 16384, 100, 1000000
NC, NS = 2, 16
NW = NC * NS
S = B // NW
K = F * S
LANES = 16
NCHK = 8
CS = S // NCHK
CE = CS * F
SB = CS // LANES
CE_H = 1536
CE_S = CE - CE_H

_Q = np.arange(CE)
_PTAB = (((_Q // F) << 7) | (_Q % F)).astype(np.int32)


def _wide_body(idx_hbm, w_hbm, b_hbm, ptab_hbm, out_hbm,
               idx2d_v, idx_v, vals_v, out_v, b_v, ptab_v,
               w_sh, isem0, isem1, hsem0, hsem1, ssem0, ssem1, wsem, tsem):
    c = lax.axis_index("c")
    s = lax.axis_index("s")
    wid = s * NC + c
    row0 = wid * S

    wcopy = pltpu.make_async_copy(w_hbm, w_sh, wsem)

    @pl.when(s == 0)
    def _():
        wcopy.start()

    pltpu.async_copy(ptab_hbm, ptab_v, tsem).wait()

    isems = (isem0, isem1)
    hsems = (hsem0, hsem1)
    ssems = (ssem0, ssem1)

    def fire_stage(chunk):
        p = chunk % 2
        return pltpu.async_copy(
            idx_hbm.at[pl.ds(row0 + chunk * CS, CS), :],
            idx2d_v.at[pl.ds(p * CS, CS), :], isems[p])

    def flatten(chunk):
        p = chunk % 2

        def body(j, _):
            for u in range(2):
                o = (2 * j + u) * LANES
                pt = ptab_v[pl.ds(o, LANES)]
                rows = lax.shift_right_logical(pt, 7) + p * CS
                cols = lax.bitwise_and(pt, 127)
                v = plsc.load_gather(idx2d_v, [rows, cols])
                idx_v[pl.ds(p * CE + o, LANES)] = v
            return 0

        lax.fori_loop(0, CE // LANES // 2, body, 0)

    def fire_h(chunk):
        p = chunk % 2
        return pltpu.async_copy(
            w_hbm.at[idx_v.at[pl.ds(p * CE, CE_H)]],
            vals_v.at[pl.ds(p * CE, CE_H)], hsems[p])

    def fire_s(chunk):
        p = chunk % 2
        return pltpu.async_copy(
            w_sh.at[idx_v.at[pl.ds(p * CE + CE_H, CE_S)]],
            vals_v.at[pl.ds(p * CE + CE_H, CE_S)], ssems[p])

    lane_f = lax.iota(jnp.int32, LANES) * F

    def reduce_chunk(chunk):
        p = chunk % 2

        def body(f, carry):
            accs, idxvs = carry
            accs = tuple(accs[i] + plsc.load_gather(vals_v, [idxvs[i]])
                         for i in range(SB))
            idxvs = tuple(iv + 1 for iv in idxvs)
            return (accs, idxvs)

        init = (tuple(jnp.zeros((LANES,), jnp.float32) for _ in range(SB)),
                tuple(lane_f + (p * CS + i * LANES) * F for i in range(SB)))
        accs, _ = lax.fori_loop(0, F, body, init)
        bvec = b_v[...]
        for b_i in range(SB):
            z = accs[b_i] + bvec
            z = jnp.clip(z, -35.0, 35.0)
            y = 1.0 / (1.0 + jnp.exp(-z))
            out_v[pl.ds(chunk * CS + b_i * LANES, LANES)] = y

    pltpu.sync_copy(b_hbm, b_v)
    stage_pend = [fire_stage(0), fire_stage(1)]
    stage_pend[0].wait()
    flatten(0)
    gh0 = fire_h(0)

    @pl.when(s == 0)
    def _():
        wcopy.wait()
    plsc.subcore_barrier()

    g_pend = [None, None]
    g_pend[0] = (gh0, fire_s(0))
    for chunk in range(NCHK):
        p = chunk % 2
        if chunk + 1 < NCHK:
            stage_pend[(chunk + 1) % 2].wait()
            flatten(chunk + 1)
            if chunk + 2 < NCHK:
                stage_pend[p] = fire_stage(chunk + 2)
            g_pend[(chunk + 1) % 2] = (fire_h(chunk + 1), fire_s(chunk + 1))
        for d in g_pend[p]:
            d.wait()
        reduce_chunk(chunk)

    pltpu.sync_copy(out_v, out_hbm.at[pl.ds(wid * S, S)])


@jax.jit
def _wide_forward(idx, w, b_arr):
    mesh = plsc.VectorSubcoreMesh(core_axis_name="c", subcore_axis_name="s")
    return pl.kernel(
        _wide_body,
        out_type=jax.ShapeDtypeStruct((B,), jnp.float32),
        mesh=mesh,
        scratch_types=[
            pltpu.VMEM((2 * CS, F), jnp.int32),
            pltpu.VMEM((2 * CE,), jnp.int32),
            pltpu.VMEM((2 * CE,), jnp.float32),
            pltpu.VMEM((S,), jnp.float32),
            pltpu.VMEM((LANES,), jnp.float32),
            pltpu.VMEM((CE,), jnp.int32),
            pltpu.VMEM_SHARED((D,), jnp.float32),
            pltpu.SemaphoreType.DMA,
            pltpu.SemaphoreType.DMA,
            pltpu.SemaphoreType.DMA,
            pltpu.SemaphoreType.DMA,
            pltpu.SemaphoreType.DMA,
            pltpu.SemaphoreType.DMA,
            pltpu.SemaphoreType.DMA,
            pltpu.SemaphoreType.DMA,
        ],
        compiler_params=pltpu.CompilerParams(
            needs_layout_passes=False, use_tc_tiling_on_sc=True),
    )(idx, w, b_arr, jnp.asarray(_PTAB))


def kernel(X_w_indices, X_d, y_pred, y, w, b):
    idx = X_w_indices.astype(jnp.int32)
    b_arr = jnp.broadcast_to(b.astype(jnp.float32), (LANES,))
    return _wide_forward(idx, w, b_arr)


# speedup vs baseline: 1.0643x; 1.0391x over previous
import jax
import jax.numpy as jnp
from jax import lax
from jax.experimental import pallas as pl
from jax.experimental.pallas import tpu as pltpu
from jax.experimental.pallas import tpu_sc as plsc

B, F, D = 16384, 100, 1000000
NC, NS = 2, 16
NW = NC * NS
S = B // NW
K = F * S
LANES = 16
NCHK = 8
CS = S // NCHK
CE = CS * F
SB = CS // LANES
CE_H = 2048
CE_S = CE - CE_H


def _wide_body(idx_hbm, w_hbm, b_hbm, out_hbm, idx_v, vals_v, out_v, b_v,
               w_sh, isem0, isem1, hsem0, hsem1, ssem0, ssem1, wsem):
    c = lax.axis_index("c")
    s = lax.axis_index("s")
    wid = s * NC + c
    row = idx_hbm.at[wid]

    wcopy = pltpu.make_async_copy(w_hbm, w_sh, wsem)

    @pl.when(s == 0)
    def _():
        wcopy.start()

    isems = (isem0, isem1)
    hsems = (hsem0, hsem1)
    ssems = (ssem0, ssem1)

    def fire_idx(chunk):
        p = chunk % 2
        return pltpu.async_copy(row.at[pl.ds(chunk * CE, CE)],
                                idx_v.at[pl.ds(p * CE, CE)], isems[p])

    def fire_h(chunk):
        p = chunk % 2
        return pltpu.async_copy(
            w_hbm.at[idx_v.at[pl.ds(p * CE, CE_H)]],
            vals_v.at[pl.ds(p * CE, CE_H)], hsems[p])

    def fire_s(chunk):
        p = chunk % 2
        return pltpu.async_copy(
            w_sh.at[idx_v.at[pl.ds(p * CE + CE_H, CE_S)]],
            vals_v.at[pl.ds(p * CE + CE_H, CE_S)], ssems[p])

    lane_f = lax.iota(jnp.int32, LANES) * F

    def reduce_chunk(chunk):
        p = chunk % 2

        def body(f, carry):
            accs, idxvs = carry
            accs = tuple(accs[i] + plsc.load_gather(vals_v, [idxvs[i]])
                         for i in range(SB))
            idxvs = tuple(iv + 1 for iv in idxvs)
            return (accs, idxvs)

        init = (tuple(jnp.zeros((LANES,), jnp.float32) for _ in range(SB)),
                tuple(lane_f + (p * CS + i * LANES) * F for i in range(SB)))
        accs, _ = lax.fori_loop(0, F, body, init)
        bvec = b_v[...]
        for b_i in range(SB):
            z = accs[b_i] + bvec
            z = jnp.clip(z, -35.0, 35.0)
            y = 1.0 / (1.0 + jnp.exp(-z))
            out_v[pl.ds(chunk * CS + b_i * LANES, LANES)] = y

    pltpu.sync_copy(b_hbm, b_v)
    idx_pend = [fire_idx(0), fire_idx(1)]
    idx_pend[0].wait()
    gh0 = fire_h(0)

    @pl.when(s == 0)
    def _():
        wcopy.wait()
    plsc.subcore_barrier()

    g_pend = [None, None]
    g_pend[0] = (gh0, fire_s(0))
    for chunk in range(NCHK):
        p = chunk % 2
        if chunk + 1 < NCHK:
            idx_pend[(chunk + 1) % 2].wait()
            if chunk + 2 < NCHK:
                idx_pend[p] = fire_idx(chunk + 2)
            g_pend[(chunk + 1) % 2] = (fire_h(chunk + 1), fire_s(chunk + 1))
        for d in g_pend[p]:
            d.wait()
        reduce_chunk(chunk)

    pltpu.sync_copy(out_v, out_hbm.at[pl.ds(wid * S, S)])


@jax.jit
def _wide_forward(idx, w, b_arr):
    mesh = plsc.VectorSubcoreMesh(core_axis_name="c", subcore_axis_name="s")
    return pl.kernel(
        _wide_body,
        out_type=jax.ShapeDtypeStruct((B,), jnp.float32),
        mesh=mesh,
        scratch_types=[
            pltpu.VMEM((2 * CE,), jnp.int32),
            pltpu.VMEM((2 * CE,), jnp.float32),
            pltpu.VMEM((S,), jnp.float32),
            pltpu.VMEM((LANES,), jnp.float32),
            pltpu.VMEM_SHARED((D,), jnp.float32),
            pltpu.SemaphoreType.DMA,
            pltpu.SemaphoreType.DMA,
            pltpu.SemaphoreType.DMA,
            pltpu.SemaphoreType.DMA,
            pltpu.SemaphoreType.DMA,
            pltpu.SemaphoreType.DMA,
            pltpu.SemaphoreType.DMA,
        ],
        compiler_params=pltpu.CompilerParams(needs_layout_passes=False),
    )(idx, w, b_arr)


def kernel(X_w_indices, X_d, y_pred, y, w, b):
    idx = X_w_indices.astype(jnp.int32).reshape(NW, K)
    b_arr = jnp.broadcast_to(b.astype(jnp.float32), (LANES,))
    return _wide_forward(idx, w, b_arr)
